# Initial kernel scaffold; baseline (speedup 1.0000x reference)
import jax
import jax.numpy as jnp
from jax.experimental import pallas as pl


def kernel(gen_matrix):
    raise NotImplementedError("write your pallas kernel here")


# speedup vs baseline: 1.1503x; 1.1503x over previous
import numpy as np
import jax
import jax.numpy as jnp
from jax.experimental import pallas as pl

_SZ = 4096

_K0 = np.uint32(0x375F238F)
_K1 = np.uint32(0xCDDB151D)
_K2 = np.uint32(int(_K0) ^ int(_K1) ^ 0x1BD11BDA)
_ROT = ((13, 15, 26, 6), (17, 29, 16, 24))
_KEYS = ((_K1, _K2), (_K2, _K0), (_K0, _K1), (_K1, _K2), (_K2, _K0))


def _bits(x1):
    x0 = jnp.full_like(x1, _K0)
    x1 = x1 + _K1
    for i, (ka, kb) in enumerate(_KEYS):
        for r in _ROT[i % 2]:
            x0 = x0 + x1
            x1 = (x1 << r) | (x1 >> (32 - r))
            x1 = x0 ^ x1
        x0 = x0 + ka
        x1 = x1 + kb + np.uint32(i + 1)
    return x0 ^ x1


def _w(bits):
    f = jax.lax.bitcast_convert_type(
        (bits >> 9) | np.uint32(0x3F800000), jnp.float32) - 1.0
    return -jnp.log(jnp.maximum(f, 1e-20))


def _gumbel_kernel(l0_ref, l1_ref, o_ref):
    i = pl.program_id(0)
    br, c = o_ref.shape
    row = jax.lax.broadcasted_iota(jnp.uint32, (br, c), 0)
    col = jax.lax.broadcasted_iota(jnp.uint32, (br, c), 1)
    base = (row + (i * br).astype(jnp.uint32)) * np.uint32(2 * c) + col * np.uint32(2)
    w0 = _w(_bits(base))
    w1 = _w(_bits(base + np.uint32(1)))
    t = jnp.exp(l0_ref[...] - l1_ref[...])
    o_ref[...] = jnp.where(w0 <= w1 * t, jnp.float32(1.0), jnp.float32(0.0))


def kernel(gen_matrix):
    l0 = gen_matrix[:, :, 0]
    l1 = gen_matrix[:, :, 1]
    br = 128
    return pl.pallas_call(
        _gumbel_kernel,
        grid=(_SZ // br,),
        in_specs=[
            pl.BlockSpec((br, _SZ), lambda i: (i, 0)),
            pl.BlockSpec((br, _SZ), lambda i: (i, 0)),
        ],
        out_specs=pl.BlockSpec((br, _SZ), lambda i: (i, 0)),
        out_shape=jax.ShapeDtypeStruct((_SZ, _SZ), jnp.float32),
    )(l0, l1)
